# Initial kernel scaffold; baseline (speedup 1.0000x reference)
import jax
import jax.numpy as jnp
from jax.experimental import pallas as pl


def kernel(drp_edge_index, drp_edge_val, pk_edge_index, pk_edge_val, edge_embeds1, ini_embeds, fnl_embeds, withdraw_rate, enc_W, enc_b, dec_W, dec_b, mlp_W, mlp_b):
    raise NotImplementedError("write your pallas kernel here")


# speedup vs baseline: 2.6308x; 2.6308x over previous
import functools

import jax
import jax.numpy as jnp
from jax import lax
from jax.experimental import pallas as pl
from jax.experimental.pallas import tpu as pltpu
from jax.experimental.pallas import tpu_sc as plsc

N = 10000
E = 320000
D = 128
LEAKY = 0.5

NCORE = 2
NSUB = 16
NW = NCORE * NSUB
CHUNK = 128
SUP = 8
EDGES_PER_W = E // NW
ITERS = 80
NSUPS = ITERS // SUP
EDGES_PAD = ITERS * CHUNK
ACC_ROWS = 10240
WB_ROWS = 624

_GDN = lax.GatherDimensionNumbers(offset_dims=(), collapsed_slice_dims=(0,),
                                  start_index_map=(0,))


def _lane_bcast(vec, t):
    idx = jnp.full((16, 1), t, jnp.int32)
    return lax.gather(vec, idx, _GDN, (1,),
                      mode=lax.GatherScatterMode.PROMISE_IN_BOUNDS)


def _pad_edges(edge_index, edge_val):
    r = edge_index[0].reshape(NW, EDGES_PER_W)
    c = edge_index[1].reshape(NW, EDGES_PER_W)
    v = edge_val.reshape(NW, EDGES_PER_W)
    pad = EDGES_PAD - EDGES_PER_W
    rp = jnp.concatenate([r, jnp.full((NW, pad), N, jnp.int32)], 1)
    cp = jnp.concatenate([c, jnp.zeros((NW, pad), jnp.int32)], 1)
    vp = jnp.concatenate([v, jnp.zeros((NW, pad), jnp.float32)], 1)
    return (rp.reshape(NW * ITERS, CHUNK), cp.reshape(NW * ITERS, CHUNK),
            vp.reshape(NW * ITERS, CHUNK))


@functools.lru_cache(maxsize=None)
def _sc_spmm(n_a, n_p):
    nio = n_a + n_p
    mesh = plsc.VectorSubcoreMesh(core_axis_name="c", subcore_axis_name="s",
                                  num_cores=NCORE)
    out_type = [jax.ShapeDtypeStruct((NCORE, N, D), jnp.float32)] * nio
    scratch = [
        pltpu.VMEM((2, SUP, CHUNK), jnp.int32),
        pltpu.VMEM((2, SUP, CHUNK), jnp.int32),
        pltpu.VMEM((2, SUP, CHUNK), jnp.float32),
        [pltpu.VMEM((CHUNK, D), jnp.float32) for _ in range(2)],
        pltpu.VMEM_SHARED((ACC_ROWS, D), jnp.float32),
        pltpu.SemaphoreType.DMA((2,)),
        pltpu.SemaphoreType.DMA((2,)),
        pltpu.SemaphoreType.DMA((2,)),
    ]

    def body(*refs):
        (a_row, a_col, a_val, p_row, p_col, p_val) = refs[:6]
        ins = refs[6:6 + nio]
        outs = refs[6 + nio:6 + 2 * nio]
        col_b, row_b, val_b, bufs, acc, semg, sems, semi = refs[6 + 2 * nio:]
        c = lax.axis_index("c")
        s = lax.axis_index("s")
        wid = c * NSUB + s

        for o in range(nio):
            erow, ecol, eval_ = (a_row, a_col, a_val) if o < n_a else (
                p_row, p_col, p_val)
            src = ins[o]
            my_base = wid * ITERS

            def idx_copies(sup, par):
                sb = pl.multiple_of(my_base + sup * SUP, 8)
                sl = pl.ds(sb, SUP)
                return (
                    pltpu.make_async_copy(ecol.at[sl], col_b.at[par],
                                          semi.at[par]),
                    pltpu.make_async_copy(erow.at[sl], row_b.at[par],
                                          semi.at[par]),
                    pltpu.make_async_copy(eval_.at[sl], val_b.at[par],
                                          semi.at[par]),
                )

            def gather_copy(p, k, b):
                return pltpu.make_async_copy(src.at[col_b.at[p, k]], bufs[b],
                                             semg.at[b])

            def scatter_start(p, k, b):
                pltpu.async_copy(bufs[b], acc.at[row_b.at[p, k]],
                                 sems.at[b], add=True)

            def scatter_wait(p, k, b):
                pltpu.make_async_copy(bufs[b], acc.at[row_b.at[p, k]],
                                      sems.at[b]).wait()

            def zero_buf(e, _):
                for j in range(D // 16):
                    bufs[0][e, pl.ds(16 * j, 16)] = jnp.zeros((16,),
                                                              jnp.float32)
                return 0
            lax.fori_loop(0, CHUNK, zero_buf, 0)
            nz = ACC_ROWS // NSUB // CHUNK
            for k in range(nz):
                pltpu.sync_copy(
                    bufs[0], acc.at[pl.ds((s * nz + k) * CHUNK, CHUNK)])
            plsc.subcore_barrier()

            for d in idx_copies(0, 0):
                d.start()
                d.wait()
            gather_copy(0, 0, 0).start()

            def pair_step(h, _):
                for bb in range(2):
                    i = h * 2 + bb
                    k = lax.rem(i, SUP)
                    p = lax.rem(i // SUP, 2)
                    pn = lax.rem(p + 1, 2)
                    b, b1 = bb, 1 - bb

                    @pl.when(jnp.logical_and(k == 0, i < (NSUPS - 1) * SUP))
                    def _():
                        for d in idx_copies(i // SUP + 1, pn):
                            d.start()

                    @pl.when(jnp.logical_and(i > 0, i < ITERS - 1))
                    def _():
                        im = i - 1
                        scatter_wait(lax.rem(im // SUP, 2), lax.rem(im, SUP),
                                     b1)

                    @pl.when(jnp.logical_and(i < ITERS - 1, k == SUP - 1))
                    def _():
                        for d in idx_copies(i // SUP + 1, pn):
                            d.wait()
                        gather_copy(pn, 0, b1).start()

                    @pl.when(jnp.logical_and(i < ITERS - 1, k < SUP - 1))
                    def _():
                        gather_copy(p, k + 1, b1).start()

                    gather_copy(p, k, b).wait()

                    def scale(g, _):
                        vvec = val_b[p, k, pl.ds(g * 16, 16)]

                        def scale4(q, _):
                            for t4 in range(4):
                                t = q * 4 + t4
                                bc = _lane_bcast(vvec, t)
                                e = g * 16 + t
                                for j in range(D // 16):
                                    sl = pl.ds(16 * j, 16)
                                    bufs[b][e, sl] = bufs[b][e, sl] * bc
                            return 0
                        lax.fori_loop(0, 4, scale4, 0)
                        return 0
                    lax.fori_loop(0, CHUNK // 16, scale, 0)

                    scatter_start(p, k, b)
                return 0
            lax.fori_loop(0, ITERS // 2, pair_step, 0)

            scatter_wait(1, SUP - 2, 0)
            scatter_wait(1, SUP - 1, 1)
            plsc.subcore_barrier()

            wb = pl.multiple_of(s * WB_ROWS, 8)
            pltpu.sync_copy(acc.at[pl.ds(wb, WB_ROWS)],
                            outs[o].at[c, pl.ds(wb, WB_ROWS)])
            tail_rows = N - NSUB * WB_ROWS

            @pl.when(s == NSUB - 1)
            def _():
                pltpu.sync_copy(
                    acc.at[pl.ds(NSUB * WB_ROWS, tail_rows)],
                    outs[o].at[c, pl.ds(NSUB * WB_ROWS, tail_rows)])
            plsc.subcore_barrier()

    return pl.kernel(body, out_type=out_type, mesh=mesh,
                     scratch_types=scratch)


_BC = 2000


def _row_spec(n=1):
    if n == 1:
        return pl.BlockSpec((_BC, D), lambda i: (i, 0))
    return pl.BlockSpec((NCORE, _BC, D), lambda i: (0, i, 0))


def _full(shape):
    return pl.BlockSpec(shape, lambda i: (0,) * len(shape))


def _leaky(x):
    return jnp.where(x >= 0, x, LEAKY * x)


def _combine1(t1p, w1p, h1p, e1p):
    def body(t_ref, w_ref, h_ref, e_ref, to_ref, wo_ref, ho_ref, eo_ref):
        to_ref[...] = t_ref[0] + t_ref[1]
        wo_ref[...] = w_ref[0] + w_ref[1]
        ho_ref[...] = h_ref[0] + h_ref[1]
        eo_ref[...] = e_ref[0] + e_ref[1]

    out = jax.ShapeDtypeStruct((N, D), jnp.float32)
    return pl.pallas_call(
        body, grid=(N // _BC,),
        in_specs=[_row_spec(2)] * 4,
        out_specs=[_row_spec()] * 4,
        out_shape=[out] * 4,
    )(t1p, w1p, h1p, e1p)


def _combine2(t2p, w2p, h2p, e2p, x0, t1):
    def body(t_ref, w_ref, h_ref, e_ref, x0_ref, t1_ref,
             inf_ref, ee_ref, w2_ref):
        inf_ref[...] = (h_ref[0] + h_ref[1]) - (e_ref[0] + e_ref[1])
        ee_ref[...] = x0_ref[...] + t1_ref[...] + t_ref[0] + t_ref[1]
        w2_ref[...] = w_ref[0] + w_ref[1]

    out = jax.ShapeDtypeStruct((N, D), jnp.float32)
    return pl.pallas_call(
        body, grid=(N // _BC,),
        in_specs=[_row_spec(2)] * 4 + [_row_spec()] * 2,
        out_specs=[_row_spec()] * 3,
        out_shape=[out] * 3,
    )(t2p, w2p, h2p, e2p, x0, t1)


def _combine3(g1p, e1p):
    def body(g_ref, e_ref, go_ref, eo_ref):
        go_ref[...] = _leaky(g_ref[0] + g_ref[1])
        eo_ref[...] = e_ref[0] + e_ref[1]

    out = jax.ShapeDtypeStruct((N, D), jnp.float32)
    return pl.pallas_call(
        body, grid=(N // _BC,),
        in_specs=[_row_spec(2)] * 2,
        out_specs=[_row_spec()] * 2,
        out_shape=[out] * 2,
    )(g1p, e1p)


def _tail(g2p, e2p, w2, ini, enc_W, enc_b, dec_W, dec_b, mW0, mb0, mW1, mb1):
    def body(g_ref, e_ref, w_ref, ini_ref, encW_ref, encb_ref, decW_ref,
             decb_ref, m0_ref, b0_ref, m1_ref, b1_ref, out_ref):
        h = _leaky(g_ref[0] + g_ref[1])
        z = jnp.dot(h, encW_ref[...], preferred_element_type=jnp.float32)
        z = z + encb_ref[...]
        zp = jnp.dot(z, decW_ref[...], preferred_element_type=jnp.float32)
        zp = zp + decb_ref[...]
        d = -w_ref[...] + (e_ref[0] + e_ref[1]) + zp
        d = jnp.dot(d, m0_ref[...], preferred_element_type=jnp.float32)
        d = _leaky(d + b0_ref[...])
        d = jnp.dot(d, m1_ref[...], preferred_element_type=jnp.float32)
        d = _leaky(d + b1_ref[...])
        out_ref[...] = ini_ref[...] + d

    return pl.pallas_call(
        body, grid=(N // _BC,),
        in_specs=[_row_spec(2), _row_spec(2), _row_spec(), _row_spec(),
                  _full((D, 64)), _full((1, 64)), _full((64, D)),
                  _full((1, D)), _full((D, D)), _full((1, D)),
                  _full((D, D)), _full((1, D))],
        out_specs=_row_spec(),
        out_shape=jax.ShapeDtypeStruct((N, D), jnp.float32),
    )(g2p, e2p, w2, ini, enc_W, enc_b.reshape(1, 64), dec_W,
      dec_b.reshape(1, D), mW0, mb0.reshape(1, D), mW1, mb1.reshape(1, D))


def kernel(drp_edge_index, drp_edge_val, pk_edge_index, pk_edge_val,
           edge_embeds1, ini_embeds, fnl_embeds, withdraw_rate,
           enc_W, enc_b, dec_W, dec_b, mlp_W, mlp_b):
    a = _pad_edges(drp_edge_index, drp_edge_val)
    p = _pad_edges(pk_edge_index, pk_edge_val)
    x0 = edge_embeds1
    w0 = fnl_embeds * withdraw_rate

    k31 = _sc_spmm(3, 1)
    k11 = _sc_spmm(1, 1)

    t1p, w1p, h1p, e1p = k31(*a, *p, x0, w0, ini_embeds, ini_embeds)
    t1, w1, h1, e1 = _combine1(t1p, w1p, h1p, e1p)
    t2p, w2p, h2p, e2p = k31(*a, *p, t1, w1, h1, e1)
    influence, edge_embed, w2 = _combine2(t2p, w2p, h2p, e2p, x0, t1)
    g1p, u1p = k11(*a, *p, influence, edge_embed)
    g1, u1 = _combine3(g1p, u1p)
    g2p, u2p = k11(*a, *p, g1, u1)
    return _tail(g2p, u2p, w2, ini_embeds, enc_W, enc_b, dec_W, dec_b,
                 mlp_W[0], mlp_b[0, 0], mlp_W[1], mlp_b[1, 0])
